# baseline (speedup 1.0000x reference)
import jax
import jax.numpy as jnp
from jax.experimental import pallas as pl
from jax.experimental.pallas import tpu as pltpu

E = 8
K = 2
D = 768
T = 8192
BT = 2048

_DN = (((1,), (1,)), ((), ()))


def _moe_body(x_ref, wg_ref, we_ref, be_ref, out_ref, w_ref):
    e = pl.program_id(1)

    @pl.when(e == 0)
    def _route():
        x = x_ref[...]
        logits = jax.lax.dot_general(
            x, wg_ref[...], _DN, preferred_element_type=jnp.float32
        )
        iota = jax.lax.broadcasted_iota(jnp.int32, (BT, E), 1)
        v1 = jnp.max(logits, axis=1, keepdims=True)
        i1 = jnp.min(jnp.where(logits == v1, iota, E), axis=1, keepdims=True)
        oh1 = iota == i1
        masked = jnp.where(oh1, -jnp.inf, logits)
        v2 = jnp.max(masked, axis=1, keepdims=True)
        i2 = jnp.min(jnp.where(masked == v2, iota, E), axis=1, keepdims=True)
        oh2 = iota == i2
        t = jnp.exp(v2 - v1)
        denom = 1.0 + t
        w = jnp.where(oh1, 1.0 / denom, 0.0) + jnp.where(oh2, t / denom, 0.0)
        for ee in range(E):
            w_ref[ee] = w[:, ee : ee + 1]

    x = x_ref[...]
    y = jax.lax.dot_general(
        x, we_ref[0], _DN, preferred_element_type=jnp.float32
    )
    term = w_ref[e] * (y + be_ref[0])

    @pl.when(e == 0)
    def _init():
        out_ref[...] = term

    @pl.when(e > 0)
    def _accum():
        out_ref[...] = out_ref[...] + term


@jax.jit
def _moe(inputs, wg, we, be):
    return pl.pallas_call(
        _moe_body,
        grid=(T // BT, E),
        in_specs=[
            pl.BlockSpec((BT, D), lambda i, e: (i, 0)),
            pl.BlockSpec((E, D), lambda i, e: (0, 0)),
            pl.BlockSpec((1, D, D), lambda i, e: (e, 0, 0)),
            pl.BlockSpec((1, 1, D), lambda i, e: (e, 0, 0)),
        ],
        out_specs=pl.BlockSpec((BT, D), lambda i, e: (i, 0)),
        out_shape=jax.ShapeDtypeStruct((T, D), jnp.float32),
        scratch_shapes=[pltpu.VMEM((E, BT, 1), jnp.float32)],
        compiler_params=pltpu.CompilerParams(vmem_limit_bytes=100 * 1024 * 1024),
    )(inputs, wg, we, be)


def kernel(inputs, Wg, We, be):
    return _moe(inputs, Wg, We, be.reshape(E, 1, D))


# speedup vs baseline: 1.4004x; 1.4004x over previous
import jax
import jax.numpy as jnp
from jax.experimental import pallas as pl

E = 8
K = 2
D = 768
T = 8192
BT = 1024

_DN = (((1,), (1,)), ((), ()))


def _moe_body(x_ref, wg_ref, we_ref, be_ref, out_ref):
    x = x_ref[...]
    logits = jax.lax.dot_general(
        x, wg_ref[...], _DN, preferred_element_type=jnp.float32
    )
    iota = jax.lax.broadcasted_iota(jnp.int32, (BT, E), 1)
    v1 = jnp.max(logits, axis=1, keepdims=True)
    i1 = jnp.min(jnp.where(logits == v1, iota, E), axis=1, keepdims=True)
    oh1 = iota == i1
    masked = jnp.where(oh1, -jnp.inf, logits)
    v2 = jnp.max(masked, axis=1, keepdims=True)
    i2 = jnp.min(jnp.where(masked == v2, iota, E), axis=1, keepdims=True)
    oh2 = iota == i2
    t = jnp.exp(v2 - v1)
    denom = 1.0 + t
    w = jnp.where(oh1, 1.0 / denom, 0.0) + jnp.where(oh2, t / denom, 0.0)

    acc = jnp.zeros((BT, D), dtype=jnp.float32)
    for e in range(E):
        y = jax.lax.dot_general(
            x, we_ref[e], _DN, preferred_element_type=jnp.float32
        )
        acc = acc + w[:, e : e + 1] * (y + be_ref[e][None, :])
    out_ref[...] = acc


@jax.jit
def _moe(inputs, wg, we, be):
    grid = T // BT
    return pl.pallas_call(
        _moe_body,
        grid=(grid,),
        in_specs=[
            pl.BlockSpec((BT, D), lambda i: (i, 0)),
            pl.BlockSpec((E, D), lambda i: (0, 0)),
            pl.BlockSpec((E, D, D), lambda i: (0, 0, 0)),
            pl.BlockSpec((E, D), lambda i: (0, 0)),
        ],
        out_specs=pl.BlockSpec((BT, D), lambda i: (i, 0)),
        out_shape=jax.ShapeDtypeStruct((T, D), jnp.float32),
    )(inputs, wg, we, be)


def kernel(inputs, Wg, We, be):
    return _moe(inputs, Wg, We, be)
